# baseline (speedup 1.0000x reference)
import jax
import jax.numpy as jnp
from jax.experimental import pallas as pl
from jax.experimental.pallas import tpu as pltpu

N = 5000
K = 12544
H = 1024
BM = 200
BK = 1792
OUT_W = 128


def _boxhead_kernel(x_ref, w1_ref, b1_ref, w2_ref, b2_ref, wh_ref, bh_ref,
                    out_ref, w1b_ref, w2b_ref, whb_ref, acc_ref):
    m = pl.program_id(0)
    k = pl.program_id(1)
    nk = pl.num_programs(1)
    ks = pl.ds(k * BK, BK)

    @pl.when(m == 0)
    def _convert_w1():
        w1b_ref[ks, :] = w1_ref[...].astype(jnp.bfloat16)

    @pl.when(jnp.logical_and(m == 0, k == 0))
    def _convert_w2():
        w2b_ref[...] = w2_ref[...].astype(jnp.bfloat16)
        whb_ref[...] = wh_ref[...].astype(jnp.bfloat16)

    part = jnp.dot(x_ref[...].astype(jnp.bfloat16), w1b_ref[ks, :],
                   preferred_element_type=jnp.float32)

    @pl.when(k == 0)
    def _init():
        acc_ref[...] = part

    @pl.when(k > 0)
    def _acc():
        acc_ref[...] += part

    @pl.when(k == nk - 1)
    def _tail():
        h1 = jnp.maximum(acc_ref[...] + b1_ref[...], 0.0)
        h2 = jnp.dot(h1.astype(jnp.bfloat16), w2b_ref[...],
                     preferred_element_type=jnp.float32)
        h2 = jnp.maximum(h2 + b2_ref[...], 0.0)
        out = jnp.dot(h2.astype(jnp.bfloat16), whb_ref[...],
                      preferred_element_type=jnp.float32)
        out_ref[...] = out + bh_ref[...]


def kernel(feature_vectors, W1, b1, W2, b2, Wc, bc, Wr, br):
    n_heads = Wc.shape[1] + Wr.shape[1]
    wh = jnp.concatenate(
        [Wc, Wr, jnp.zeros((H, OUT_W - n_heads), dtype=Wc.dtype)], axis=1)
    bh = jnp.concatenate(
        [bc, br, jnp.zeros((OUT_W - n_heads,), dtype=bc.dtype)])

    nk = K // BK
    grid = (N // BM, nk)
    out = pl.pallas_call(
        _boxhead_kernel,
        grid=grid,
        in_specs=[
            pl.BlockSpec((BM, BK), lambda m, k: (m, k)),
            pl.BlockSpec((BK, H), lambda m, k: (jnp.where(m == 0, k, nk - 1), 0)),
            pl.BlockSpec((1, H), lambda m, k: (0, 0)),
            pl.BlockSpec((H, H), lambda m, k: (0, 0)),
            pl.BlockSpec((1, H), lambda m, k: (0, 0)),
            pl.BlockSpec((H, OUT_W), lambda m, k: (0, 0)),
            pl.BlockSpec((1, OUT_W), lambda m, k: (0, 0)),
        ],
        out_specs=pl.BlockSpec((BM, OUT_W), lambda m, k: (m, 0)),
        out_shape=jax.ShapeDtypeStruct((N, OUT_W), jnp.float32),
        scratch_shapes=[
            pltpu.VMEM((K, H), jnp.bfloat16),
            pltpu.VMEM((H, H), jnp.bfloat16),
            pltpu.VMEM((H, OUT_W), jnp.bfloat16),
            pltpu.VMEM((BM, H), jnp.float32),
        ],
        compiler_params=pltpu.CompilerParams(
            dimension_semantics=("arbitrary", "arbitrary"),
        ),
    )(feature_vectors, W1, b1.reshape(1, H), W2, b2.reshape(1, H),
      wh, bh.reshape(1, OUT_W))

    return out[:, :Wc.shape[1]], out[:, Wc.shape[1]:n_heads]


# speedup vs baseline: 1.5039x; 1.5039x over previous
import jax
import jax.numpy as jnp
from jax.experimental import pallas as pl
from jax.experimental.pallas import tpu as pltpu

N = 5000
K = 12544
H = 1024
BM = 1000
BKC = 1792
NKC = K // BKC
NM = N // BM
TAIL_CHUNK = 200
OUT_W = 128


def _boxhead_kernel(x_hbm, w1_hbm, b1_ref, w2_ref, b2_ref, wh_ref, bh_ref,
                    out_ref, xbuf, w1stage, w1b, acc_ref, xsem, wsem):
    m = pl.program_id(0)

    def x_copy(r, s, buf):
        return pltpu.make_async_copy(
            x_hbm.at[pl.ds(r * BM, BM), pl.ds(s * BKC, BKC)],
            xbuf.at[buf], xsem.at[buf])

    def w1_copy(j):
        return pltpu.make_async_copy(
            w1_hbm.at[pl.ds(j * BKC, BKC), :], w1stage, wsem)

    @pl.when(m == 0)
    def _kickoff():
        x_copy(0, 0, 0).start()
        w1_copy(0).start()

    for j in range(NKC):
        buf = (m * NKC + j) % 2
        nbuf = 1 - buf

        if j < NKC - 1:
            x_copy(m, j + 1, nbuf).start()
        else:
            @pl.when(m < NM - 1)
            def _pf():
                x_copy(m + 1, 0, nbuf).start()

        @pl.when(m == 0)
        def _convert():
            w1_copy(j).wait()
            w1b[pl.ds(j * BKC, BKC), :] = w1stage[...].astype(jnp.bfloat16)
            if j < NKC - 1:
                w1_copy(j + 1).start()

        x_copy(m, j, buf).wait()
        part = jnp.dot(xbuf[buf].astype(jnp.bfloat16),
                       w1b[pl.ds(j * BKC, BKC), :],
                       preferred_element_type=jnp.float32)
        if j == 0:
            acc_ref[...] = part
        else:
            acc_ref[...] += part

    for t in range(BM // TAIL_CHUNK):
        rows = pl.ds(t * TAIL_CHUNK, TAIL_CHUNK)
        h1 = jnp.maximum(acc_ref[rows, :] + b1_ref[...], 0.0)
        h2 = jnp.dot(h1.astype(jnp.bfloat16), w2_ref[...],
                     preferred_element_type=jnp.float32)
        h2 = jnp.maximum(h2 + b2_ref[...], 0.0)
        out = jnp.dot(h2.astype(jnp.bfloat16), wh_ref[...],
                      preferred_element_type=jnp.float32)
        out_ref[rows, :] = out + bh_ref[...]


def kernel(feature_vectors, W1, b1, W2, b2, Wc, bc, Wr, br):
    n_heads = Wc.shape[1] + Wr.shape[1]
    wh = jnp.concatenate(
        [Wc, Wr, jnp.zeros((H, OUT_W - n_heads), dtype=Wc.dtype)], axis=1)
    bh = jnp.concatenate(
        [bc, br, jnp.zeros((OUT_W - n_heads,), dtype=bc.dtype)])

    w2b = W2.astype(jnp.bfloat16)
    whb = wh.astype(jnp.bfloat16)

    grid = (NM,)
    out = pl.pallas_call(
        _boxhead_kernel,
        grid=grid,
        in_specs=[
            pl.BlockSpec(memory_space=pl.ANY),
            pl.BlockSpec(memory_space=pl.ANY),
            pl.BlockSpec((1, H), lambda m: (0, 0)),
            pl.BlockSpec((H, H), lambda m: (0, 0)),
            pl.BlockSpec((1, H), lambda m: (0, 0)),
            pl.BlockSpec((H, OUT_W), lambda m: (0, 0)),
            pl.BlockSpec((1, OUT_W), lambda m: (0, 0)),
        ],
        out_specs=pl.BlockSpec((BM, OUT_W), lambda m: (m, 0)),
        out_shape=jax.ShapeDtypeStruct((N, OUT_W), jnp.float32),
        scratch_shapes=[
            pltpu.VMEM((2, BM, BKC), jnp.float32),
            pltpu.VMEM((BKC, H), jnp.float32),
            pltpu.VMEM((K, H), jnp.bfloat16),
            pltpu.VMEM((BM, H), jnp.float32),
            pltpu.SemaphoreType.DMA((2,)),
            pltpu.SemaphoreType.DMA,
        ],
        compiler_params=pltpu.CompilerParams(
            dimension_semantics=("arbitrary",),
            vmem_limit_bytes=67108864,
        ),
    )(feature_vectors, W1, b1.reshape(1, H), w2b, b2.reshape(1, H),
      whb, bh.reshape(1, OUT_W))

    return out[:, :Wc.shape[1]], out[:, Wc.shape[1]:n_heads]


# speedup vs baseline: 1.5055x; 1.0010x over previous
import jax
import jax.numpy as jnp
from jax.experimental import pallas as pl
from jax.experimental.pallas import tpu as pltpu

N = 5000
K = 12544
H = 1024
BM = 1000
BKC = 1792
NKC = K // BKC
NM = N // BM
TAIL_CHUNK = 200
OUT_W = 128


def _boxhead_kernel(x_hbm, w1_hbm, b1_ref, w2_ref, b2_ref, wh_ref, bh_ref,
                    out_ref, xbuf, w1stage, w1b, acc_ref, xsem, wsem):
    m = pl.program_id(0)

    half = BKC // 2

    def x_copy_pair(r, s, buf):
        return (
            pltpu.make_async_copy(
                x_hbm.at[pl.ds(r * BM, BM), pl.ds(s * BKC, half)],
                xbuf.at[buf, :, pl.ds(0, half)], xsem.at[buf, 0]),
            pltpu.make_async_copy(
                x_hbm.at[pl.ds(r * BM, BM), pl.ds(s * BKC + half, half)],
                xbuf.at[buf, :, pl.ds(half, half)], xsem.at[buf, 1]),
        )

    def x_start(r, s, buf):
        for c in x_copy_pair(r, s, buf):
            c.start()

    def x_wait(r, s, buf):
        for c in x_copy_pair(r, s, buf):
            c.wait()

    def w1_copy(j):
        return pltpu.make_async_copy(
            w1_hbm.at[pl.ds(j * BKC, BKC), :], w1stage, wsem)

    @pl.when(m == 0)
    def _kickoff():
        x_start(0, 0, 0)
        w1_copy(0).start()

    for j in range(NKC):
        buf = (m * NKC + j) % 2
        nbuf = 1 - buf

        if j < NKC - 1:
            x_start(m, j + 1, nbuf)
        else:
            @pl.when(m < NM - 1)
            def _pf():
                x_start(m + 1, 0, nbuf)

        @pl.when(m == 0)
        def _convert():
            w1_copy(j).wait()
            w1b[pl.ds(j * BKC, BKC), :] = w1stage[...].astype(jnp.bfloat16)
            if j < NKC - 1:
                w1_copy(j + 1).start()

        x_wait(m, j, buf)
        part = jnp.dot(xbuf[buf].astype(jnp.bfloat16),
                       w1b[pl.ds(j * BKC, BKC), :],
                       preferred_element_type=jnp.float32)
        if j == 0:
            acc_ref[...] = part
        else:
            acc_ref[...] += part

    for t in range(BM // TAIL_CHUNK):
        rows = pl.ds(t * TAIL_CHUNK, TAIL_CHUNK)
        h1 = jnp.maximum(acc_ref[rows, :] + b1_ref[...], 0.0)
        h2 = jnp.dot(h1.astype(jnp.bfloat16), w2_ref[...],
                     preferred_element_type=jnp.float32)
        h2 = jnp.maximum(h2 + b2_ref[...], 0.0)
        out = jnp.dot(h2.astype(jnp.bfloat16), wh_ref[...],
                      preferred_element_type=jnp.float32)
        out_ref[rows, :] = out + bh_ref[...]


def kernel(feature_vectors, W1, b1, W2, b2, Wc, bc, Wr, br):
    n_heads = Wc.shape[1] + Wr.shape[1]
    wh = jnp.concatenate(
        [Wc, Wr, jnp.zeros((H, OUT_W - n_heads), dtype=Wc.dtype)], axis=1)
    bh = jnp.concatenate(
        [bc, br, jnp.zeros((OUT_W - n_heads,), dtype=bc.dtype)])

    w2b = W2.astype(jnp.bfloat16)
    whb = wh.astype(jnp.bfloat16)

    grid = (NM,)
    out = pl.pallas_call(
        _boxhead_kernel,
        grid=grid,
        in_specs=[
            pl.BlockSpec(memory_space=pl.ANY),
            pl.BlockSpec(memory_space=pl.ANY),
            pl.BlockSpec((1, H), lambda m: (0, 0)),
            pl.BlockSpec((H, H), lambda m: (0, 0)),
            pl.BlockSpec((1, H), lambda m: (0, 0)),
            pl.BlockSpec((H, OUT_W), lambda m: (0, 0)),
            pl.BlockSpec((1, OUT_W), lambda m: (0, 0)),
        ],
        out_specs=pl.BlockSpec((BM, OUT_W), lambda m: (m, 0)),
        out_shape=jax.ShapeDtypeStruct((N, OUT_W), jnp.float32),
        scratch_shapes=[
            pltpu.VMEM((2, BM, BKC), jnp.float32),
            pltpu.VMEM((BKC, H), jnp.float32),
            pltpu.VMEM((K, H), jnp.bfloat16),
            pltpu.VMEM((BM, H), jnp.float32),
            pltpu.SemaphoreType.DMA((2, 2)),
            pltpu.SemaphoreType.DMA,
        ],
        compiler_params=pltpu.CompilerParams(
            dimension_semantics=("arbitrary",),
            vmem_limit_bytes=67108864,
        ),
    )(feature_vectors, W1, b1.reshape(1, H), w2b, b2.reshape(1, H),
      whb, bh.reshape(1, OUT_W))

    return out[:, :Wc.shape[1]], out[:, Wc.shape[1]:n_heads]
